# baseline (speedup 1.0000x reference)
import functools

import jax
import jax.numpy as jnp
from jax.experimental import pallas as pl
from jax.experimental.pallas import tpu as pltpu

_K = 3277
_V = 32768
_ROWS = 8
_MININT = -2147483648


def _rowsum(mask):
    ones = jnp.ones((mask.shape[1], 1), jnp.float32)
    return jax.lax.dot_general(
        mask.astype(jnp.float32),
        ones,
        (((1,), (0,)), ((), ())),
        preferred_element_type=jnp.float32,
    )


def _topk_mask_body(x_ref, o_ref):
    x = x_ref[...]
    b = jax.lax.bitcast_convert_type(x, jnp.int32)
    s = jnp.where(b < 0, b ^ jnp.int32(0x7FFFFFFF), b)

    def val_step(i, t_u):
        bit = jnp.left_shift(jnp.int32(1), jnp.int32(31) - i)
        cand_u = t_u | bit
        cand_s = cand_u ^ jnp.int32(_MININT)
        cnt = _rowsum(s >= cand_s)
        return jnp.where(cnt >= jnp.float32(_K), cand_u, t_u)

    t_u = jax.lax.fori_loop(
        0, 32, val_step, jnp.zeros((x.shape[0], 1), jnp.int32)
    )
    t_s = t_u ^ jnp.int32(_MININT)

    gt = s > t_s
    eq = s == t_s
    n_gt = _rowsum(gt)
    cnt_eq = _rowsum(eq)
    need = jnp.float32(_K) - n_gt

    idx = jax.lax.broadcasted_iota(jnp.int32, x.shape, 1)

    def tie_search():
        def idx_step(i, res):
            bit = jnp.left_shift(jnp.int32(1), jnp.int32(15) - i)
            cand = res | bit
            g = _rowsum(eq & (idx < cand))
            return jnp.where(g <= need, cand, res)

        return jax.lax.fori_loop(
            0, 16, idx_step, jnp.zeros((x.shape[0], 1), jnp.int32)
        )

    res = jax.lax.cond(
        jnp.any(cnt_eq != need),
        tie_search,
        lambda: jnp.full((x.shape[0], 1), 65535, jnp.int32),
    )

    keep = gt | (eq & (idx < res))
    o_ref[...] = jnp.where(keep, x, jnp.float32(-jnp.inf))


@functools.partial(jax.jit, static_argnums=())
def _topk_mask(flat):
    n_rows = flat.shape[0]
    return pl.pallas_call(
        _topk_mask_body,
        grid=(n_rows // _ROWS,),
        in_specs=[pl.BlockSpec((_ROWS, _V), lambda i: (i, 0))],
        out_specs=pl.BlockSpec((_ROWS, _V), lambda i: (i, 0)),
        out_shape=jax.ShapeDtypeStruct((n_rows, _V), jnp.float32),
        compiler_params=pltpu.CompilerParams(
            dimension_semantics=("parallel",),
        ),
    )(flat)


def kernel(logits, k):
    B, S, V = logits.shape
    out = _topk_mask(logits.reshape(B * S, V))
    return out.reshape(B, S, V)


# speedup vs baseline: 3.5562x; 3.5562x over previous
import functools

import jax
import jax.numpy as jnp
from jax.experimental import pallas as pl
from jax.experimental.pallas import tpu as pltpu

_K = 3277
_V = 32768
_ROWS = 8
_MININT = -2147483648


def _rowsum(mask):
    return jnp.sum(mask.astype(jnp.int32), axis=1, keepdims=True)


def _topk_mask_body(x_ref, o_ref):
    x = x_ref[...]
    b = jax.lax.bitcast_convert_type(x, jnp.int32)
    s = jnp.where(b < 0, b ^ jnp.int32(0x7FFFFFFF), b)

    def val_step(i, t_u):
        bit = jnp.left_shift(jnp.int32(1), jnp.int32(31) - i)
        cand_u = t_u | bit
        cand_s = cand_u ^ jnp.int32(_MININT)
        cnt = _rowsum(s >= cand_s)
        return jnp.where(cnt >= _K, cand_u, t_u)

    t_u = jax.lax.fori_loop(
        0, 32, val_step, jnp.zeros((x.shape[0], 1), jnp.int32)
    )
    t_s = t_u ^ jnp.int32(_MININT)

    gt = s > t_s
    eq = s == t_s
    n_gt = _rowsum(gt)
    cnt_eq = _rowsum(eq)
    need = _K - n_gt

    idx = jax.lax.broadcasted_iota(jnp.int32, x.shape, 1)

    def tie_search():
        def idx_step(i, res):
            bit = jnp.left_shift(jnp.int32(1), jnp.int32(15) - i)
            cand = res | bit
            g = _rowsum(eq & (idx < cand))
            return jnp.where(g <= need, cand, res)

        return jax.lax.fori_loop(
            0, 16, idx_step, jnp.zeros((x.shape[0], 1), jnp.int32)
        )

    res = jax.lax.cond(
        jnp.any(cnt_eq != need),
        tie_search,
        lambda: jnp.full((x.shape[0], 1), 65535, jnp.int32),
    )

    keep = gt | (eq & (idx < res))
    o_ref[...] = jnp.where(keep, x, jnp.float32(-jnp.inf))


@functools.partial(jax.jit, static_argnums=())
def _topk_mask(flat):
    n_rows = flat.shape[0]
    return pl.pallas_call(
        _topk_mask_body,
        grid=(n_rows // _ROWS,),
        in_specs=[pl.BlockSpec((_ROWS, _V), lambda i: (i, 0))],
        out_specs=pl.BlockSpec((_ROWS, _V), lambda i: (i, 0)),
        out_shape=jax.ShapeDtypeStruct((n_rows, _V), jnp.float32),
        compiler_params=pltpu.CompilerParams(
            dimension_semantics=("parallel",),
        ),
    )(flat)


def kernel(logits, k):
    B, S, V = logits.shape
    out = _topk_mask(logits.reshape(B * S, V))
    return out.reshape(B, S, V)


# speedup vs baseline: 6.1492x; 1.7292x over previous
import functools

import jax
import jax.numpy as jnp
from jax.experimental import pallas as pl
from jax.experimental.pallas import tpu as pltpu

_K = 3277
_V = 32768
_ROWS = 16
_MININT = -2147483648


def _rowsum(mask):
    v = mask.astype(jnp.int32)
    while v.shape[1] > 128:
        h = v.shape[1] // 2
        v = v[:, :h] + v[:, h:]
    return jnp.sum(v, axis=1, keepdims=True)


def _topk_mask_body(x_ref, o_ref):
    x = x_ref[...]
    b = jax.lax.bitcast_convert_type(x, jnp.int32)
    s = jnp.where(b < 0, b ^ jnp.int32(0x7FFFFFFF), b)

    def val_step(i, t_u):
        bit = jnp.left_shift(jnp.int32(1), jnp.int32(31) - i)
        cand_u = t_u | bit
        cand_s = cand_u ^ jnp.int32(_MININT)
        cnt = _rowsum(s >= cand_s)
        return jnp.where(cnt >= _K, cand_u, t_u)

    t_u = jax.lax.fori_loop(
        0, 32, val_step, jnp.zeros((x.shape[0], 1), jnp.int32)
    )
    t_s = t_u ^ jnp.int32(_MININT)

    gt = s > t_s
    eq = s == t_s
    n_gt = _rowsum(gt)
    cnt_eq = _rowsum(eq)
    need = _K - n_gt

    idx = jax.lax.broadcasted_iota(jnp.int32, x.shape, 1)

    def tie_search():
        def idx_step(i, res):
            bit = jnp.left_shift(jnp.int32(1), jnp.int32(15) - i)
            cand = res | bit
            g = _rowsum(eq & (idx < cand))
            return jnp.where(g <= need, cand, res)

        return jax.lax.fori_loop(
            0, 16, idx_step, jnp.zeros((x.shape[0], 1), jnp.int32)
        )

    res = jax.lax.cond(
        jnp.any(cnt_eq != need),
        tie_search,
        lambda: jnp.full((x.shape[0], 1), 65535, jnp.int32),
    )

    keep = gt | (eq & (idx < res))
    o_ref[...] = jnp.where(keep, x, jnp.float32(-jnp.inf))


@functools.partial(jax.jit, static_argnums=())
def _topk_mask(flat):
    n_rows = flat.shape[0]
    return pl.pallas_call(
        _topk_mask_body,
        grid=(n_rows // _ROWS,),
        in_specs=[pl.BlockSpec((_ROWS, _V), lambda i: (i, 0))],
        out_specs=pl.BlockSpec((_ROWS, _V), lambda i: (i, 0)),
        out_shape=jax.ShapeDtypeStruct((n_rows, _V), jnp.float32),
        compiler_params=pltpu.CompilerParams(
            dimension_semantics=("parallel",),
        ),
    )(flat)


def kernel(logits, k):
    B, S, V = logits.shape
    out = _topk_mask(logits.reshape(B * S, V))
    return out.reshape(B, S, V)


# speedup vs baseline: 6.2637x; 1.0186x over previous
import functools

import jax
import jax.numpy as jnp
from jax.experimental import pallas as pl
from jax.experimental.pallas import tpu as pltpu

_K = 3277
_V = 32768
_ROWS = 16
_MININT = -2147483648


def _rowsum(mask):
    v = mask.astype(jnp.int32)
    while v.shape[1] > 128:
        h = v.shape[1] // 2
        v = v[:, :h] + v[:, h:]
    return jnp.sum(v, axis=1, keepdims=True)


def _topk_mask_body(x_ref, o_ref):
    x = x_ref[...]
    b = jax.lax.bitcast_convert_type(x, jnp.int32)
    s = jnp.where(b < 0, b ^ jnp.int32(0x7FFFFFFF), b)

    def val_step(i, carry):
        t_u, n_ge = carry
        bit = jnp.left_shift(jnp.int32(1), jnp.int32(31) - i)
        cand_u = t_u | bit
        cand_s = cand_u ^ jnp.int32(_MININT)
        cnt = _rowsum(s >= cand_s)
        acc = cnt >= _K
        return jnp.where(acc, cand_u, t_u), jnp.where(acc, cnt, n_ge)

    r = x.shape[0]
    t_u, n_ge = jax.lax.fori_loop(
        0,
        32,
        val_step,
        (jnp.zeros((r, 1), jnp.int32), jnp.full((r, 1), _V, jnp.int32)),
    )
    t_s = t_u ^ jnp.int32(_MININT)

    def tie_path():
        gt = s > t_s
        eq = s == t_s
        cnt_eq = _rowsum(eq)
        need = _K - (n_ge - cnt_eq)
        idx = jax.lax.broadcasted_iota(jnp.int32, x.shape, 1)

        def idx_step(i, res):
            bit = jnp.left_shift(jnp.int32(1), jnp.int32(15) - i)
            cand = res | bit
            g = _rowsum(eq & (idx < cand))
            return jnp.where(g <= need, cand, res)

        res = jax.lax.fori_loop(
            0, 16, idx_step, jnp.zeros((r, 1), jnp.int32)
        )
        keep = gt | (eq & (idx < res))
        return jnp.where(keep, x, jnp.float32(-jnp.inf))

    def fast_path():
        return jnp.where(s >= t_s, x, jnp.float32(-jnp.inf))

    o_ref[...] = jax.lax.cond(jnp.any(n_ge != _K), tie_path, fast_path)


@functools.partial(jax.jit, static_argnums=())
def _topk_mask(flat):
    n_rows = flat.shape[0]
    return pl.pallas_call(
        _topk_mask_body,
        grid=(n_rows // _ROWS,),
        in_specs=[pl.BlockSpec((_ROWS, _V), lambda i: (i, 0))],
        out_specs=pl.BlockSpec((_ROWS, _V), lambda i: (i, 0)),
        out_shape=jax.ShapeDtypeStruct((n_rows, _V), jnp.float32),
        compiler_params=pltpu.CompilerParams(
            dimension_semantics=("parallel",),
        ),
    )(flat)


def kernel(logits, k):
    B, S, V = logits.shape
    out = _topk_mask(logits.reshape(B * S, V))
    return out.reshape(B, S, V)


# speedup vs baseline: 7.9679x; 1.2721x over previous
import functools

import jax
import jax.numpy as jnp
from jax.experimental import pallas as pl
from jax.experimental.pallas import tpu as pltpu

_K = 3277
_V = 32768
_ROWS = 16
_MININT = -2147483648


_CHUNK = 1024


def _lane_reduce(v):
    while v.shape[1] > 128:
        h = v.shape[1] // 2
        v = v[:, :h] + v[:, h:]
    return jnp.sum(v, axis=1, keepdims=True)


def _count_pred(s_ref, pred):
    r = s_ref.shape[0]
    acc = jnp.zeros((r, _CHUNK), jnp.int32)
    for c in range(_V // _CHUNK):
        sl = s_ref[:, c * _CHUNK : (c + 1) * _CHUNK]
        acc = acc + pred(sl, c).astype(jnp.int32)
    return _lane_reduce(acc)


def _topk_mask_body(x_ref, o_ref, s_ref):
    b = jax.lax.bitcast_convert_type(x_ref[...], jnp.int32)
    s_ref[...] = jnp.where(b < 0, b ^ jnp.int32(0x7FFFFFFF), b)
    r = x_ref.shape[0]

    def val_step(i, carry):
        t_u, n_ge = carry
        bit = jnp.left_shift(jnp.int32(1), jnp.int32(31) - i)
        cand_u = t_u | bit
        cand_s = cand_u ^ jnp.int32(_MININT)
        cnt = _count_pred(s_ref, lambda sl, c: sl >= cand_s)
        acc = cnt >= _K
        return jnp.where(acc, cand_u, t_u), jnp.where(acc, cnt, n_ge)

    t_u, n_ge = jax.lax.fori_loop(
        0,
        32,
        val_step,
        (jnp.zeros((r, 1), jnp.int32), jnp.full((r, 1), _V, jnp.int32)),
    )
    t_s = t_u ^ jnp.int32(_MININT)

    def tie_path():
        cnt_eq = _count_pred(s_ref, lambda sl, c: sl == t_s)
        need = _K - (n_ge - cnt_eq)

        def idx_step(i, res):
            bit = jnp.left_shift(jnp.int32(1), jnp.int32(15) - i)
            cand = res | bit

            def pred(sl, c):
                idx = jax.lax.broadcasted_iota(
                    jnp.int32, sl.shape, 1
                ) + jnp.int32(c * _CHUNK)
                return (sl == t_s) & (idx < cand)

            g = _count_pred(s_ref, pred)
            return jnp.where(g <= need, cand, res)

        res = jax.lax.fori_loop(
            0, 16, idx_step, jnp.zeros((r, 1), jnp.int32)
        )
        s = s_ref[...]
        idx = jax.lax.broadcasted_iota(jnp.int32, s.shape, 1)
        keep = (s > t_s) | ((s == t_s) & (idx < res))
        return jnp.where(keep, x_ref[...], jnp.float32(-jnp.inf))

    def fast_path():
        return jnp.where(
            s_ref[...] >= t_s, x_ref[...], jnp.float32(-jnp.inf)
        )

    o_ref[...] = jax.lax.cond(jnp.any(n_ge != _K), tie_path, fast_path)


@functools.partial(jax.jit, static_argnums=())
def _topk_mask(flat):
    n_rows = flat.shape[0]
    return pl.pallas_call(
        _topk_mask_body,
        grid=(n_rows // _ROWS,),
        in_specs=[pl.BlockSpec((_ROWS, _V), lambda i: (i, 0))],
        out_specs=pl.BlockSpec((_ROWS, _V), lambda i: (i, 0)),
        out_shape=jax.ShapeDtypeStruct((n_rows, _V), jnp.float32),
        scratch_shapes=[pltpu.VMEM((_ROWS, _V), jnp.int32)],
        compiler_params=pltpu.CompilerParams(
            dimension_semantics=("parallel",),
        ),
    )(flat)


def kernel(logits, k):
    B, S, V = logits.shape
    out = _topk_mask(logits.reshape(B * S, V))
    return out.reshape(B, S, V)


# speedup vs baseline: 8.8002x; 1.1045x over previous
import functools

import jax
import jax.numpy as jnp
from jax.experimental import pallas as pl
from jax.experimental.pallas import tpu as pltpu

_K = 3277
_V = 32768
_ROWS = 32
_MININT = -2147483648


_CHUNK = 1024


def _lane_reduce(v):
    while v.shape[1] > 128:
        h = v.shape[1] // 2
        v = v[:, :h] + v[:, h:]
    return jnp.sum(v, axis=1, keepdims=True)


def _count_pred(s_ref, pred):
    r = s_ref.shape[0]
    acc = jnp.zeros((r, _CHUNK), jnp.int32)
    for c in range(_V // _CHUNK):
        sl = s_ref[:, c * _CHUNK : (c + 1) * _CHUNK]
        acc = acc + pred(sl, c).astype(jnp.int32)
    return _lane_reduce(acc)


def _topk_mask_body(x_ref, o_ref, s_ref):
    b = jax.lax.bitcast_convert_type(x_ref[...], jnp.int32)
    s_ref[...] = jnp.where(b < 0, b ^ jnp.int32(0x7FFFFFFF), b)
    r = x_ref.shape[0]

    def val_step(i, carry):
        t_u, n_ge = carry
        bit = jnp.left_shift(jnp.int32(1), jnp.int32(31) - i)
        cand_u = t_u | bit
        cand_s = cand_u ^ jnp.int32(_MININT)
        cnt = _count_pred(s_ref, lambda sl, c: sl >= cand_s)
        acc = cnt >= _K
        return jnp.where(acc, cand_u, t_u), jnp.where(acc, cnt, n_ge)

    t_u, n_ge = jax.lax.fori_loop(
        0,
        32,
        val_step,
        (jnp.zeros((r, 1), jnp.int32), jnp.full((r, 1), _V, jnp.int32)),
    )
    t_s = t_u ^ jnp.int32(_MININT)

    def tie_path():
        cnt_eq = _count_pred(s_ref, lambda sl, c: sl == t_s)
        need = _K - (n_ge - cnt_eq)

        def idx_step(i, res):
            bit = jnp.left_shift(jnp.int32(1), jnp.int32(15) - i)
            cand = res | bit

            def pred(sl, c):
                idx = jax.lax.broadcasted_iota(
                    jnp.int32, sl.shape, 1
                ) + jnp.int32(c * _CHUNK)
                return (sl == t_s) & (idx < cand)

            g = _count_pred(s_ref, pred)
            return jnp.where(g <= need, cand, res)

        res = jax.lax.fori_loop(
            0, 16, idx_step, jnp.zeros((r, 1), jnp.int32)
        )
        s = s_ref[...]
        idx = jax.lax.broadcasted_iota(jnp.int32, s.shape, 1)
        keep = (s > t_s) | ((s == t_s) & (idx < res))
        return jnp.where(keep, x_ref[...], jnp.float32(-jnp.inf))

    def fast_path():
        return jnp.where(
            s_ref[...] >= t_s, x_ref[...], jnp.float32(-jnp.inf)
        )

    o_ref[...] = jax.lax.cond(jnp.any(n_ge != _K), tie_path, fast_path)


@functools.partial(jax.jit, static_argnums=())
def _topk_mask(flat):
    n_rows = flat.shape[0]
    return pl.pallas_call(
        _topk_mask_body,
        grid=(n_rows // _ROWS,),
        in_specs=[pl.BlockSpec((_ROWS, _V), lambda i: (i, 0))],
        out_specs=pl.BlockSpec((_ROWS, _V), lambda i: (i, 0)),
        out_shape=jax.ShapeDtypeStruct((n_rows, _V), jnp.float32),
        scratch_shapes=[pltpu.VMEM((_ROWS, _V), jnp.int32)],
        compiler_params=pltpu.CompilerParams(
            dimension_semantics=("parallel",),
        ),
    )(flat)


def kernel(logits, k):
    B, S, V = logits.shape
    out = _topk_mask(logits.reshape(B * S, V))
    return out.reshape(B, S, V)
